# Initial kernel scaffold; baseline (speedup 1.0000x reference)
import jax
import jax.numpy as jnp
from jax.experimental import pallas as pl


def kernel(x, ei, ea, batch, y, W_msg, W_edge, W_self, b):
    raise NotImplementedError("write your pallas kernel here")


# speedup vs baseline: 1.8503x; 1.8503x over previous
import functools

import numpy as np
import jax
import jax.numpy as jnp
from jax import lax
from jax.experimental import pallas as pl
from jax.experimental.pallas import tpu as pltpu
from jax.experimental.pallas import tpu_sc as plsc

N_NODES = 10000
N_EDGES = 320000
D = 128
D_EDGE = 16
NB = 4
S = 2500

NC = 2
NS = 16
NW = NC * NS
CH = 80
S_PAD = 2560
N_PAD = 10112
ROWS_PER_TILE = N_PAD // NS
DUMMY_DST = N_PAD - 8


def _sc_mesh():
    return plsc.VectorSubcoreMesh(
        core_axis_name="c", subcore_axis_name="s", num_cores=NC, num_subcores=NS
    )


def _gather_rows(table, idx3, n_chunks, d, dtype):
    rows_per_w = n_chunks * CH
    b_total = NW * rows_per_w

    @functools.partial(
        pl.kernel,
        out_type=jax.ShapeDtypeStruct((b_total, d), dtype),
        mesh=_sc_mesh(),
        scratch_types=[
            pltpu.VMEM((n_chunks, CH), jnp.int32),
            pltpu.VMEM((rows_per_w, d), dtype),
            pltpu.SemaphoreType.DMA,
        ],
    )
    def gk(table_hbm, idx_hbm, out_hbm, idx_v, rows_v, sem):
        wid = lax.axis_index("s") * NC + lax.axis_index("c")
        pltpu.sync_copy(idx_hbm.at[wid], idx_v)
        for j in range(n_chunks):
            pltpu.async_copy(
                table_hbm.at[idx_v.at[j]], rows_v.at[pl.ds(j * CH, CH)], sem
            ).wait()
        pltpu.sync_copy(rows_v, out_hbm.at[pl.ds(wid * rows_per_w, rows_per_w)])

    return gk(table, idx3)


def _scatter_block(u, msg4, dst3):
    @functools.partial(
        pl.kernel,
        out_type=jax.ShapeDtypeStruct((N_PAD, D), jnp.float32),
        mesh=_sc_mesh(),
        scratch_types=[
            pltpu.VMEM_SHARED((N_PAD, D), jnp.float32),
            pltpu.VMEM((2, CH), jnp.int32),
            pltpu.VMEM((2, CH, D), jnp.float32),
        ],
    )
    def sk(u_hbm, msg_hbm, dst_hbm, p_hbm, acc_sh, idx_v, msg_v):
        cid = lax.axis_index("c")
        sid = lax.axis_index("s")
        r0 = sid * ROWS_PER_TILE

        @pl.when(cid == 0)
        def _init():
            pltpu.sync_copy(
                u_hbm.at[pl.ds(r0, ROWS_PER_TILE)],
                acc_sh.at[pl.ds(r0, ROWS_PER_TILE)],
            )
            pltpu.sync_copy(dst_hbm.at[sid], idx_v)
            pltpu.sync_copy(msg_hbm.at[sid], msg_v)

        plsc.subcore_barrier()

        @pl.when(cid == 0)
        def _scatter():
            for j in range(2):
                pltpu.sync_copy(msg_v.at[j], acc_sh.at[idx_v.at[j]], add=True)

        plsc.subcore_barrier()

        @pl.when(cid == 0)
        def _writeback():
            pltpu.sync_copy(
                acc_sh.at[pl.ds(r0, ROWS_PER_TILE)],
                p_hbm.at[pl.ds(r0, ROWS_PER_TILE)],
            )

    return sk(u, msg4, dst3)


def _mm_block(g, e, p, wmsg, wself, b2, use_relu):
    def body(g_ref, e_ref, p_ref, wm_ref, ws_ref, b_ref, msg_out, u_out):
        gg = g_ref[...]
        pp = p_ref[...]
        if use_relu:
            gg = jnp.maximum(gg, 0.0)
            pp = jnp.maximum(pp, 0.0)
        msg_out[...] = (
            jnp.dot(gg, wm_ref[...], preferred_element_type=jnp.float32)
            + e_ref[...]
        )
        u_out[...] = (
            jnp.dot(pp, ws_ref[...], preferred_element_type=jnp.float32)
            + b_ref[...]
        )

    return pl.pallas_call(
        body,
        out_shape=[
            jax.ShapeDtypeStruct((S_PAD, D), jnp.float32),
            jax.ShapeDtypeStruct((N_PAD, D), jnp.float32),
        ],
    )(g, e, p, wmsg, wself, b2)


def _edge_mm(ea_rows, wedge):
    def body(a_ref, w_ref, out_ref):
        out_ref[...] = jnp.dot(
            a_ref[...], w_ref[...], preferred_element_type=jnp.float32
        )

    return pl.pallas_call(
        body, out_shape=jax.ShapeDtypeStruct((NB * S_PAD, D), jnp.float32)
    )(ea_rows, wedge)


def _relu_kernel(p):
    def body(p_ref, o_ref):
        o_ref[...] = jnp.maximum(p_ref[...], 0.0)

    return pl.pallas_call(
        body, out_shape=jax.ShapeDtypeStruct((N_PAD, D), jnp.float32)
    )(p)


def kernel(x, ei, ea, batch, y, W_msg, W_edge, W_self, b):
    base = np.arange(0, N_NODES, NB)
    k2 = np.stack(
        [(2 * ((base + k) % N_NODES)) % N_EDGES for k in range(1, NB + 1)]
    ).astype(np.int32)
    k2_pad = np.zeros((NB, S_PAD), np.int32)
    k2_pad[:, :S] = k2

    src = jnp.take(ei[0], jnp.asarray(k2_pad), axis=0)
    dst_real = jnp.take(ei[1], jnp.asarray(k2[:, :S]), axis=0)
    dst = jnp.concatenate(
        [dst_real, jnp.full((NB, S_PAD - S), DUMMY_DST, jnp.int32)], axis=1
    )

    src3 = src.reshape(NB, NW, 1, CH)
    dst3 = dst.reshape(NB, NS, 2, CH)

    x_pad = jnp.pad(x, ((0, N_PAD - N_NODES), (0, 0)))
    b2 = b.reshape(1, D)

    ea_rows = jnp.take(ea, jnp.asarray(k2_pad.reshape(-1)), axis=0)
    e_all = _edge_mm(ea_rows, W_edge)

    p_cur = x_pad
    for k in range(NB):
        g = _gather_rows(p_cur, src3[k], 1, D, jnp.float32)
        e_k = lax.slice(e_all, (k * S_PAD, 0), ((k + 1) * S_PAD, D))
        msg, u = _mm_block(g, e_k, p_cur, W_msg, W_self, b2, use_relu=(k > 0))
        p_cur = _scatter_block(u, msg.reshape(NS, 2, CH, D), dst3[k])

    h = _relu_kernel(p_cur)
    return h[:N_NODES]


# speedup vs baseline: 2.1265x; 1.1493x over previous
import functools

import numpy as np
import jax
import jax.numpy as jnp
from jax import lax
from jax.experimental import pallas as pl
from jax.experimental.pallas import tpu as pltpu
from jax.experimental.pallas import tpu_sc as plsc

N_NODES = 10000
N_EDGES = 320000
D = 128
D_EDGE = 16
NB = 4
S = 2500

NC = 2
NS = 16
NW = NC * NS
CH = 80
S_PAD = 2560
N_PAD = 10112
ROWS_PER_TILE = N_PAD // NS
DUMMY_DST = N_PAD - 8


def _sc_mesh():
    return plsc.VectorSubcoreMesh(
        core_axis_name="c", subcore_axis_name="s", num_cores=NC, num_subcores=NS
    )


def _gather_rows(table, idx3):
    @functools.partial(
        pl.kernel,
        out_type=jax.ShapeDtypeStruct((S_PAD, D), jnp.float32),
        mesh=_sc_mesh(),
        scratch_types=[
            pltpu.VMEM((1, CH), jnp.int32),
            pltpu.VMEM((CH, D), jnp.float32),
            pltpu.SemaphoreType.DMA,
        ],
    )
    def gk(table_hbm, idx_hbm, out_hbm, idx_v, rows_v, sem):
        wid = lax.axis_index("s") * NC + lax.axis_index("c")
        pltpu.sync_copy(idx_hbm.at[wid], idx_v)
        pltpu.async_copy(table_hbm.at[idx_v.at[0]], rows_v, sem).wait()
        pltpu.sync_copy(rows_v, out_hbm.at[pl.ds(wid * CH, CH)])

    return gk(table, idx3)


def _scatter_block(u, msg4, dst3, nsrc3):
    out_type = [jax.ShapeDtypeStruct((N_PAD, D), jnp.float32)]
    scratch = [
        pltpu.VMEM_SHARED((N_PAD, D), jnp.float32),
        pltpu.VMEM((2, CH), jnp.int32),
        pltpu.VMEM((2, CH, D), jnp.float32),
        pltpu.SemaphoreType.DMA,
    ]
    with_gather = nsrc3 is not None
    if with_gather:
        out_type.append(jax.ShapeDtypeStruct((S_PAD, D), jnp.float32))
        scratch += [
            pltpu.VMEM((2, CH), jnp.int32),
            pltpu.VMEM((2 * CH, D), jnp.float32),
            pltpu.SemaphoreType.DMA,
        ]

    @functools.partial(
        pl.kernel, out_type=out_type, mesh=_sc_mesh(), scratch_types=scratch
    )
    def sk(*refs):
        if with_gather:
            (u_hbm, msg_hbm, dst_hbm, nsrc_hbm, p_hbm, g_hbm,
             acc_sh, idx_v, msg_v, sem, nidx_v, grows_v, gsem) = refs
        else:
            (u_hbm, msg_hbm, dst_hbm, p_hbm,
             acc_sh, idx_v, msg_v, sem) = refs
        cid = lax.axis_index("c")
        sid = lax.axis_index("s")
        r0 = sid * ROWS_PER_TILE

        @pl.when(cid == 0)
        def _init():
            pltpu.sync_copy(
                u_hbm.at[pl.ds(r0, ROWS_PER_TILE)],
                acc_sh.at[pl.ds(r0, ROWS_PER_TILE)],
            )
            pltpu.sync_copy(dst_hbm.at[sid], idx_v)
            pltpu.sync_copy(msg_hbm.at[sid], msg_v)

        plsc.subcore_barrier()

        @pl.when(cid == 0)
        def _scatter():
            for j in range(2):
                pltpu.sync_copy(msg_v.at[j], acc_sh.at[idx_v.at[j]], add=True)

        plsc.subcore_barrier()

        @pl.when(cid == 0)
        def _writeback():
            wb = pltpu.async_copy(
                acc_sh.at[pl.ds(r0, ROWS_PER_TILE)],
                p_hbm.at[pl.ds(r0, ROWS_PER_TILE)],
                sem,
            )
            if with_gather:
                pltpu.sync_copy(nsrc_hbm.at[sid], nidx_v)
                for j in range(2):
                    pltpu.async_copy(
                        acc_sh.at[nidx_v.at[j]],
                        grows_v.at[pl.ds(j * CH, CH)],
                        gsem,
                    ).wait()
                pltpu.sync_copy(grows_v, g_hbm.at[pl.ds(sid * 2 * CH, 2 * CH)])
            wb.wait()

    return sk(u, msg4, dst3, nsrc3) if with_gather else sk(u, msg4, dst3)


def _mm_block(g, e, p, wmsg, wself, b2, use_relu):
    n_rows = p.shape[0]

    def body(g_ref, e_ref, p_ref, wm_ref, ws_ref, b_ref, msg_out, u_out):
        gg = g_ref[...]
        pp = p_ref[...]
        if use_relu:
            gg = jnp.maximum(gg, 0.0)
            pp = jnp.maximum(pp, 0.0)
        msg_out[...] = (
            jnp.dot(gg, wm_ref[...], preferred_element_type=jnp.float32)
            + e_ref[...]
        )
        u_out[pl.ds(0, n_rows), :] = (
            jnp.dot(pp, ws_ref[...], preferred_element_type=jnp.float32)
            + b_ref[...]
        )
        if n_rows < N_PAD:
            u_out[pl.ds(n_rows, N_PAD - n_rows), :] = jnp.zeros(
                (N_PAD - n_rows, D), jnp.float32
            )

    return pl.pallas_call(
        body,
        out_shape=[
            jax.ShapeDtypeStruct((S_PAD, D), jnp.float32),
            jax.ShapeDtypeStruct((N_PAD, D), jnp.float32),
        ],
    )(g, e, p, wmsg, wself, b2)


def _edge_mm(ea_rows, wedge):
    def body(a_ref, w_ref, out_ref):
        out_ref[...] = jnp.dot(
            a_ref[...], w_ref[...], preferred_element_type=jnp.float32
        )

    return pl.pallas_call(
        body, out_shape=jax.ShapeDtypeStruct((NB * S_PAD, D), jnp.float32)
    )(ea_rows, wedge)


def _relu_kernel(p):
    def body(p_ref, o_ref):
        o_ref[...] = jnp.maximum(p_ref[pl.ds(0, N_NODES), :], 0.0)

    return pl.pallas_call(
        body, out_shape=jax.ShapeDtypeStruct((N_NODES, D), jnp.float32)
    )(p)


def kernel(x, ei, ea, batch, y, W_msg, W_edge, W_self, b):
    base = np.arange(0, N_NODES, NB)
    k2 = np.stack(
        [(2 * ((base + k) % N_NODES)) % N_EDGES for k in range(1, NB + 1)]
    ).astype(np.int32)
    strided = np.arange(S)[None, :] * 8 + 2 * np.arange(1, NB + 1)[:, None]
    strided[NB - 1] = np.roll(strided[NB - 1] - 8, -1)
    if not np.array_equal(k2, strided):
        raise AssertionError("static edge-subset pattern mismatch")

    cols = lax.slice(ei, (0, 0), (2, 8 * S)).reshape(2, S, 8)
    ea8 = lax.slice(ea, (0, 0), (8 * S, D_EDGE)).reshape(S, 8, D_EDGE)
    srcs, dsts, ea_rows = [], [], []
    pad_i = jnp.zeros((S_PAD - S,), jnp.int32)
    pad_d = jnp.full((S_PAD - S,), DUMMY_DST, jnp.int32)
    pad_e = jnp.zeros((S_PAD - S, D_EDGE), jnp.float32)
    for k in range(1, NB + 1):
        m = 2 * k
        if m < 8:
            s_k, d_k, a_k = cols[0, :, m], cols[1, :, m], ea8[:, m]
        else:
            s_k = jnp.roll(cols[0, :, 0], -1)
            d_k = jnp.roll(cols[1, :, 0], -1)
            a_k = jnp.roll(ea8[:, 0], -1, axis=0)
        srcs.append(jnp.concatenate([s_k, pad_i]))
        dsts.append(jnp.concatenate([d_k, pad_d]))
        ea_rows.append(jnp.concatenate([a_k, pad_e], axis=0))
    src = jnp.stack(srcs)
    dst = jnp.stack(dsts)
    ea_all = jnp.concatenate(ea_rows, axis=0)

    src_w = src.reshape(NB, NW, 1, CH)
    src_t = src.reshape(NB, NS, 2, CH)
    dst3 = dst.reshape(NB, NS, 2, CH)
    b2 = b.reshape(1, D)

    e_all = _edge_mm(ea_all, W_edge)

    p_cur = x
    g = _gather_rows(x, src_w[0])
    for k in range(NB):
        e_k = lax.slice(e_all, (k * S_PAD, 0), ((k + 1) * S_PAD, D))
        msg, u = _mm_block(g, e_k, p_cur, W_msg, W_self, b2, use_relu=(k > 0))
        nsrc = src_t[k + 1] if k + 1 < NB else None
        res = _scatter_block(u, msg.reshape(NS, 2, CH, D), dst3[k], nsrc)
        if k + 1 < NB:
            p_cur, g = res
        else:
            (p_cur,) = res

    return _relu_kernel(p_cur)


# speedup vs baseline: 2.2935x; 1.0786x over previous
import functools

import numpy as np
import jax
import jax.numpy as jnp
from jax import lax
from jax.experimental import pallas as pl
from jax.experimental.pallas import tpu as pltpu
from jax.experimental.pallas import tpu_sc as plsc

N_NODES = 10000
N_EDGES = 320000
D = 128
D_EDGE = 16
NB = 4
S = 2500

NC = 2
NS = 16
NW = NC * NS
CH = 80
S_PAD = 2560
N_PAD = 10240
N_HALF = N_PAD // NC
TILE_ROWS = N_HALF // NS
ACC_ROWS = N_HALF + 16
DUMMY_GATH = N_HALF
DUMMY_SCAT = N_HALF + 8
DUMMY_DST = N_PAD - 8


def _sc_mesh():
    return plsc.VectorSubcoreMesh(
        core_axis_name="c", subcore_axis_name="s", num_cores=NC, num_subcores=NS
    )


def _clamp_to_half(idx_v, row, c0, dummy):
    for t in range(CH // 16):
        v = idx_v[row, pl.ds(t * 16, 16)]
        lv = v - c0
        ok = (lv >= 0) & (lv < N_HALF)
        idx_v[row, pl.ds(t * 16, 16)] = jnp.where(ok, lv, dummy)


def _gather_rows(table, idx3):
    @functools.partial(
        pl.kernel,
        out_type=jax.ShapeDtypeStruct((S_PAD, D), jnp.float32),
        mesh=_sc_mesh(),
        scratch_types=[
            pltpu.VMEM((1, CH), jnp.int32),
            pltpu.VMEM((CH, D), jnp.float32),
            pltpu.SemaphoreType.DMA,
        ],
    )
    def gk(table_hbm, idx_hbm, out_hbm, idx_v, rows_v, sem):
        wid = lax.axis_index("s") * NC + lax.axis_index("c")
        pltpu.sync_copy(idx_hbm.at[wid], idx_v)
        pltpu.async_copy(table_hbm.at[idx_v.at[0]], rows_v, sem).wait()
        pltpu.sync_copy(rows_v, out_hbm.at[pl.ds(wid * CH, CH)])

    return gk(table, idx3)


def _scatter_block(u, msg, dst3, nsrc3):
    out_type = [jax.ShapeDtypeStruct((N_PAD, D), jnp.float32)]
    scratch = [
        pltpu.VMEM_SHARED((ACC_ROWS, D), jnp.float32),
        pltpu.VMEM((2, CH), jnp.int32),
        pltpu.VMEM((2, CH, D), jnp.float32),
        pltpu.SemaphoreType.DMA,
    ]
    with_gather = nsrc3 is not None
    if with_gather:
        out_type += [
            jax.ShapeDtypeStruct((S_PAD, D), jnp.float32),
            jax.ShapeDtypeStruct((S_PAD, D), jnp.float32),
        ]
        scratch += [
            pltpu.VMEM((2, CH), jnp.int32),
            pltpu.VMEM((2 * CH, D), jnp.float32),
            pltpu.VMEM((8, D), jnp.float32),
            pltpu.SemaphoreType.DMA,
        ]

    @functools.partial(
        pl.kernel, out_type=out_type, mesh=_sc_mesh(), scratch_types=scratch
    )
    def sk(*refs):
        if with_gather:
            (u_hbm, msg_hbm, dst_hbm, nsrc_hbm, p_hbm, g0_hbm, g1_hbm,
             acc_sh, idx_v, msg_v, sem, nidx_v, grows_v, zbuf, gsem) = refs
        else:
            (u_hbm, msg_hbm, dst_hbm, p_hbm,
             acc_sh, idx_v, msg_v, sem) = refs
        cid = lax.axis_index("c")
        sid = lax.axis_index("s")
        c0 = cid * N_HALF
        r0 = sid * TILE_ROWS

        pltpu.sync_copy(
            u_hbm.at[pl.ds(c0 + r0, TILE_ROWS)], acc_sh.at[pl.ds(r0, TILE_ROWS)]
        )
        pltpu.sync_copy(dst_hbm.at[sid], idx_v)
        for j in range(2):
            _clamp_to_half(idx_v, j, c0, DUMMY_SCAT)
            pltpu.sync_copy(
                msg_hbm.at[pl.ds(sid * 2 * CH + j * CH, CH)], msg_v.at[j]
            )
        if with_gather:
            @pl.when(sid == 0)
            def _zero_miss_rows():
                for r in range(8):
                    for t in range(D // 16):
                        zbuf[r, pl.ds(t * 16, 16)] = jnp.zeros((16,), jnp.float32)
                pltpu.sync_copy(zbuf, acc_sh.at[pl.ds(DUMMY_GATH, 8)])

        plsc.subcore_barrier()

        for j in range(2):
            pltpu.sync_copy(msg_v.at[j], acc_sh.at[idx_v.at[j]], add=True)

        plsc.subcore_barrier()

        wb = pltpu.async_copy(
            acc_sh.at[pl.ds(r0, TILE_ROWS)],
            p_hbm.at[pl.ds(c0 + r0, TILE_ROWS)],
            sem,
        )
        if with_gather:
            pltpu.sync_copy(nsrc_hbm.at[sid], nidx_v)
            for j in range(2):
                _clamp_to_half(nidx_v, j, c0, DUMMY_GATH)
                pltpu.async_copy(
                    acc_sh.at[nidx_v.at[j]],
                    grows_v.at[pl.ds(j * CH, CH)],
                    gsem,
                ).wait()

            @pl.when(cid == 0)
            def _out0():
                pltpu.sync_copy(grows_v, g0_hbm.at[pl.ds(sid * 2 * CH, 2 * CH)])

            @pl.when(cid == 1)
            def _out1():
                pltpu.sync_copy(grows_v, g1_hbm.at[pl.ds(sid * 2 * CH, 2 * CH)])

        wb.wait()

    return sk(u, msg, dst3, nsrc3) if with_gather else sk(u, msg, dst3)


def _mm_block(g_parts, e, p, wmsg, wself, b2, use_relu):
    n_rows = p.shape[0]
    two_g = len(g_parts) == 2

    def body(*refs):
        if two_g:
            g0_ref, g1_ref, e_ref, p_ref, wm_ref, ws_ref, b_ref, msg_out, u_out = refs
            gg = g0_ref[...] + g1_ref[...]
        else:
            g_ref, e_ref, p_ref, wm_ref, ws_ref, b_ref, msg_out, u_out = refs
            gg = g_ref[...]
        pp = p_ref[...]
        if use_relu:
            gg = jnp.maximum(gg, 0.0)
            pp = jnp.maximum(pp, 0.0)
        msg_out[...] = (
            jnp.dot(gg, wm_ref[...], preferred_element_type=jnp.float32)
            + e_ref[...]
        )
        u_out[pl.ds(0, n_rows), :] = (
            jnp.dot(pp, ws_ref[...], preferred_element_type=jnp.float32)
            + b_ref[...]
        )
        if n_rows < N_PAD:
            u_out[pl.ds(n_rows, N_PAD - n_rows), :] = jnp.zeros(
                (N_PAD - n_rows, D), jnp.float32
            )

    return pl.pallas_call(
        body,
        out_shape=[
            jax.ShapeDtypeStruct((S_PAD, D), jnp.float32),
            jax.ShapeDtypeStruct((N_PAD, D), jnp.float32),
        ],
    )(*g_parts, e, p, wmsg, wself, b2)


def _edge_mm(ea_rows, wedge):
    def body(a_ref, w_ref, out_ref):
        out_ref[...] = jnp.dot(
            a_ref[...], w_ref[...], preferred_element_type=jnp.float32
        )

    return pl.pallas_call(
        body, out_shape=jax.ShapeDtypeStruct((NB * S_PAD, D), jnp.float32)
    )(ea_rows, wedge)


def _relu_kernel(p):
    def body(p_ref, o_ref):
        o_ref[...] = jnp.maximum(p_ref[pl.ds(0, N_NODES), :], 0.0)

    return pl.pallas_call(
        body, out_shape=jax.ShapeDtypeStruct((N_NODES, D), jnp.float32)
    )(p)


def kernel(x, ei, ea, batch, y, W_msg, W_edge, W_self, b):
    base = np.arange(0, N_NODES, NB)
    k2 = np.stack(
        [(2 * ((base + k) % N_NODES)) % N_EDGES for k in range(1, NB + 1)]
    ).astype(np.int32)
    strided = np.arange(S)[None, :] * 8 + 2 * np.arange(1, NB + 1)[:, None]
    strided[NB - 1] = np.roll(strided[NB - 1] - 8, -1)
    if not np.array_equal(k2, strided):
        raise AssertionError("static edge-subset pattern mismatch")

    cols = lax.slice(ei, (0, 0), (2, 8 * S)).reshape(2, S, 8)
    ea8 = lax.slice(ea, (0, 0), (8 * S, D_EDGE)).reshape(S, 8, D_EDGE)
    srcs, dsts, ea_rows = [], [], []
    pad_i = jnp.zeros((S_PAD - S,), jnp.int32)
    pad_d = jnp.full((S_PAD - S,), DUMMY_DST, jnp.int32)
    pad_e = jnp.zeros((S_PAD - S, D_EDGE), jnp.float32)
    for k in range(1, NB + 1):
        m = 2 * k
        if m < 8:
            s_k, d_k, a_k = cols[0, :, m], cols[1, :, m], ea8[:, m]
        else:
            s_k = jnp.roll(cols[0, :, 0], -1)
            d_k = jnp.roll(cols[1, :, 0], -1)
            a_k = jnp.roll(ea8[:, 0], -1, axis=0)
        srcs.append(jnp.concatenate([s_k, pad_i]))
        dsts.append(jnp.concatenate([d_k, pad_d]))
        ea_rows.append(jnp.concatenate([a_k, pad_e], axis=0))
    src = jnp.stack(srcs)
    dst = jnp.stack(dsts)
    ea_all = jnp.concatenate(ea_rows, axis=0)

    src_w = src.reshape(NB, NW, 1, CH)
    src_t = src.reshape(NB, NS, 2, CH)
    dst3 = dst.reshape(NB, NS, 2, CH)
    b2 = b.reshape(1, D)

    e_all = _edge_mm(ea_all, W_edge)

    p_cur = x
    g_parts = (_gather_rows(x, src_w[0]),)
    for k in range(NB):
        e_k = lax.slice(e_all, (k * S_PAD, 0), ((k + 1) * S_PAD, D))
        msg, u = _mm_block(g_parts, e_k, p_cur, W_msg, W_self, b2,
                           use_relu=(k > 0))
        nsrc = src_t[k + 1] if k + 1 < NB else None
        res = _scatter_block(u, msg, dst3[k], nsrc)
        if k + 1 < NB:
            p_cur, g0, g1 = res
            g_parts = (g0, g1)
        else:
            (p_cur,) = res

    return _relu_kernel(p_cur)


# speedup vs baseline: 2.8925x; 1.2612x over previous
import functools

import numpy as np
import jax
import jax.numpy as jnp
from jax import lax
from jax.experimental import pallas as pl
from jax.experimental.pallas import tpu as pltpu
from jax.experimental.pallas import tpu_sc as plsc

N_NODES = 10000
N_EDGES = 320000
D = 128
D_EDGE = 16
NB = 4
S = 2500

NC = 2
NS = 16
NW = NC * NS
CH = 80
S_PAD = 2560
N_PAD = 10240
N_HALF = N_PAD // NC
TILE_ROWS = N_HALF // NS
ACC_ROWS = N_HALF + 16
DUMMY_GATH = N_HALF
DUMMY_SCAT = N_HALF + 8
DUMMY_DST = N_PAD - 8


def _sc_mesh():
    return plsc.VectorSubcoreMesh(
        core_axis_name="c", subcore_axis_name="s", num_cores=NC, num_subcores=NS
    )


def _clamp_to_half(idx_v, row, c0, dummy):
    for t in range(CH // 16):
        v = idx_v[row, pl.ds(t * 16, 16)]
        lv = v - c0
        ok = (lv >= 0) & (lv < N_HALF)
        idx_v[row, pl.ds(t * 16, 16)] = jnp.where(ok, lv, dummy)


def _gather_rows(table, idx3):
    @functools.partial(
        pl.kernel,
        out_type=jax.ShapeDtypeStruct((S_PAD, D), jnp.float32),
        mesh=_sc_mesh(),
        scratch_types=[
            pltpu.VMEM((1, CH), jnp.int32),
            pltpu.VMEM((CH, D), jnp.float32),
            pltpu.SemaphoreType.DMA,
        ],
    )
    def gk(table_hbm, idx_hbm, out_hbm, idx_v, rows_v, sem):
        wid = lax.axis_index("s") * NC + lax.axis_index("c")
        pltpu.sync_copy(idx_hbm.at[wid], idx_v)
        pltpu.async_copy(table_hbm.at[idx_v.at[0]], rows_v, sem).wait()
        pltpu.sync_copy(rows_v, out_hbm.at[pl.ds(wid * CH, CH)])

    return gk(table, idx3)


def _scatter_block(u, msg, dst3, nsrc3):
    out_type = [jax.ShapeDtypeStruct((N_PAD, D), jnp.float32)]
    scratch = [
        pltpu.VMEM_SHARED((ACC_ROWS, D), jnp.float32),
        pltpu.VMEM((2, CH), jnp.int32),
        pltpu.VMEM((2, CH, D), jnp.float32),
        pltpu.SemaphoreType.DMA,
    ]
    with_gather = nsrc3 is not None
    if with_gather:
        out_type += [
            jax.ShapeDtypeStruct((S_PAD, D), jnp.float32),
            jax.ShapeDtypeStruct((S_PAD, D), jnp.float32),
        ]
        scratch += [
            pltpu.VMEM((2, CH), jnp.int32),
            pltpu.VMEM((2 * CH, D), jnp.float32),
            pltpu.VMEM((8, D), jnp.float32),
            pltpu.SemaphoreType.DMA,
        ]

    @functools.partial(
        pl.kernel, out_type=out_type, mesh=_sc_mesh(), scratch_types=scratch
    )
    def sk(*refs):
        if with_gather:
            (u_hbm, msg_hbm, dst_hbm, nsrc_hbm, p_hbm, g0_hbm, g1_hbm,
             acc_sh, idx_v, msg_v, sem, nidx_v, grows_v, zbuf, gsem) = refs
        else:
            (u_hbm, msg_hbm, dst_hbm, p_hbm,
             acc_sh, idx_v, msg_v, sem) = refs
        cid = lax.axis_index("c")
        sid = lax.axis_index("s")
        c0 = cid * N_HALF
        r0 = sid * TILE_ROWS

        pltpu.sync_copy(
            u_hbm.at[pl.ds(c0 + r0, TILE_ROWS)], acc_sh.at[pl.ds(r0, TILE_ROWS)]
        )
        pltpu.sync_copy(dst_hbm.at[sid], idx_v)
        for j in range(2):
            _clamp_to_half(idx_v, j, c0, DUMMY_SCAT)
            pltpu.sync_copy(
                msg_hbm.at[pl.ds(sid * 2 * CH + j * CH, CH)], msg_v.at[j]
            )
        if with_gather:
            @pl.when(sid == 0)
            def _zero_miss_rows():
                for r in range(8):
                    for t in range(D // 16):
                        zbuf[r, pl.ds(t * 16, 16)] = jnp.zeros((16,), jnp.float32)
                pltpu.sync_copy(zbuf, acc_sh.at[pl.ds(DUMMY_GATH, 8)])

        plsc.subcore_barrier()

        for j in range(2):
            pltpu.sync_copy(msg_v.at[j], acc_sh.at[idx_v.at[j]], add=True)

        plsc.subcore_barrier()

        wb = pltpu.async_copy(
            acc_sh.at[pl.ds(r0, TILE_ROWS)],
            p_hbm.at[pl.ds(c0 + r0, TILE_ROWS)],
            sem,
        )
        if with_gather:
            pltpu.sync_copy(nsrc_hbm.at[sid], nidx_v)
            for j in range(2):
                _clamp_to_half(nidx_v, j, c0, DUMMY_GATH)
                pltpu.async_copy(
                    acc_sh.at[nidx_v.at[j]],
                    grows_v.at[pl.ds(j * CH, CH)],
                    gsem,
                ).wait()

            @pl.when(cid == 0)
            def _out0():
                pltpu.sync_copy(grows_v, g0_hbm.at[pl.ds(sid * 2 * CH, 2 * CH)])

            @pl.when(cid == 1)
            def _out1():
                pltpu.sync_copy(grows_v, g1_hbm.at[pl.ds(sid * 2 * CH, 2 * CH)])

        wb.wait()

    return sk(u, msg, dst3, nsrc3) if with_gather else sk(u, msg, dst3)


def _mm_block(g_parts, e_all, k_idx, p, wmsg, wself, b2, use_relu):
    n_rows = p.shape[0]
    two_g = len(g_parts) == 2

    def body(*refs):
        if two_g:
            g0_ref, g1_ref, e_ref, p_ref, wm_ref, ws_ref, b_ref, msg_out, u_out = refs
            gg = g0_ref[...] + g1_ref[...]
        else:
            g_ref, e_ref, p_ref, wm_ref, ws_ref, b_ref, msg_out, u_out = refs
            gg = g_ref[...]
        pp = p_ref[...]
        if use_relu:
            gg = jnp.maximum(gg, 0.0)
            pp = jnp.maximum(pp, 0.0)
        msg_out[...] = (
            jnp.dot(gg, wm_ref[...], preferred_element_type=jnp.float32)
            + e_ref[...]
        )
        u_out[pl.ds(0, n_rows), :] = (
            jnp.dot(pp, ws_ref[...], preferred_element_type=jnp.float32)
            + b_ref[...]
        )
        if n_rows < N_PAD:
            u_out[pl.ds(n_rows, N_PAD - n_rows), :] = jnp.zeros(
                (N_PAD - n_rows, D), jnp.float32
            )

    in_specs = [pl.BlockSpec(gp.shape, lambda i: (0, 0)) for gp in g_parts] + [
        pl.BlockSpec((S_PAD, D), lambda i: (0, k_idx)),
        pl.BlockSpec((n_rows, D), lambda i: (0, 0)),
        pl.BlockSpec((D, D), lambda i: (0, 0)),
        pl.BlockSpec((D, D), lambda i: (0, 0)),
        pl.BlockSpec((1, D), lambda i: (0, 0)),
    ]
    return pl.pallas_call(
        body,
        grid=(1,),
        in_specs=in_specs,
        out_specs=[
            pl.BlockSpec((S_PAD, D), lambda i: (0, 0)),
            pl.BlockSpec((N_PAD, D), lambda i: (0, 0)),
        ],
        out_shape=[
            jax.ShapeDtypeStruct((S_PAD, D), jnp.float32),
            jax.ShapeDtypeStruct((N_PAD, D), jnp.float32),
        ],
    )(*g_parts, e_all, p, wmsg, wself, b2)


M_PLANES = [2, 4, 6, 0]


def _edge_mm(ea_flat, wedge):
    def body(a_ref, w_ref, out_ref):
        w = w_ref[...]
        blocks = []
        for m in M_PLANES:
            lo = D_EDGE * m
            hi = 8 * D_EDGE - lo - D_EDGE
            parts = ([jnp.zeros((lo, D), jnp.float32)] if lo else [])
            parts.append(w)
            if hi:
                parts.append(jnp.zeros((hi, D), jnp.float32))
            blocks.append(jnp.concatenate(parts, axis=0))
        wcat = jnp.concatenate(blocks, axis=1)
        out_ref[...] = jnp.dot(
            a_ref[...], wcat, preferred_element_type=jnp.float32
        )

    return pl.pallas_call(
        body, out_shape=jax.ShapeDtypeStruct((S_PAD, NB * D), jnp.float32)
    )(ea_flat, wedge)


def _relu_kernel(p):
    def body(p_ref, o_ref):
        o_ref[...] = jnp.maximum(p_ref[pl.ds(0, N_NODES), :], 0.0)

    return pl.pallas_call(
        body, out_shape=jax.ShapeDtypeStruct((N_NODES, D), jnp.float32)
    )(p)


def kernel(x, ei, ea, batch, y, W_msg, W_edge, W_self, b):
    base = np.arange(0, N_NODES, NB)
    k2 = np.stack(
        [(2 * ((base + k) % N_NODES)) % N_EDGES for k in range(1, NB + 1)]
    ).astype(np.int32)
    for ki, m in enumerate(M_PLANES):
        if not np.array_equal(np.sort(k2[ki]), np.arange(S) * 8 + m):
            raise AssertionError("static edge-subset pattern mismatch")

    cols = lax.slice(ei, (0, 0), (2, 8 * S)).reshape(2, S, 8)
    srcs, dsts = [], []
    pad_i = jnp.zeros((S_PAD - S,), jnp.int32)
    pad_d = jnp.full((S_PAD - S,), DUMMY_DST, jnp.int32)
    for m in M_PLANES:
        srcs.append(jnp.concatenate([cols[0, :, m], pad_i]))
        dsts.append(jnp.concatenate([cols[1, :, m], pad_d]))
    src = jnp.stack(srcs)
    dst = jnp.stack(dsts)

    src_w = src.reshape(NB, NW, 1, CH)
    src_t = src.reshape(NB, NS, 2, CH)
    dst3 = dst.reshape(NB, NS, 2, CH)
    b2 = b.reshape(1, D)

    ea_flat = lax.slice(ea, (0, 0), (8 * S_PAD, D_EDGE)).reshape(
        S_PAD, 8 * D_EDGE
    )
    e_all = _edge_mm(ea_flat, W_edge)

    p_cur = x
    g_parts = (_gather_rows(x, src_w[0]),)
    for k in range(NB):
        msg, u = _mm_block(g_parts, e_all, k, p_cur, W_msg, W_self, b2,
                           use_relu=(k > 0))
        nsrc = src_t[k + 1] if k + 1 < NB else None
        res = _scatter_block(u, msg, dst3[k], nsrc)
        if k + 1 < NB:
            p_cur, g0, g1 = res
            g_parts = (g0, g1)
        else:
            (p_cur,) = res

    return _relu_kernel(p_cur)


# speedup vs baseline: 3.1784x; 1.0988x over previous
import functools

import numpy as np
import jax
import jax.numpy as jnp
from jax import lax
from jax.experimental import pallas as pl
from jax.experimental.pallas import tpu as pltpu
from jax.experimental.pallas import tpu_sc as plsc

N_NODES = 10000
N_EDGES = 320000
D = 128
D_EDGE = 16
NB = 4
S = 2500

NC = 2
NS = 16
NW = NC * NS
CH = 80
S_PAD = 2560
N_PAD = 10240
N_HALF = N_PAD // NC
TILE_ROWS = N_HALF // NS
ACC_ROWS = N_HALF + 16
DUMMY_GATH = N_HALF
DUMMY_SCAT = N_HALF + 8
DUMMY_DST = N_PAD - 8


def _sc_mesh():
    return plsc.VectorSubcoreMesh(
        core_axis_name="c", subcore_axis_name="s", num_cores=NC, num_subcores=NS
    )


def _clamp_to_half(idx_v, row, c0, dummy):
    for t in range(CH // 16):
        v = idx_v[row, pl.ds(t * 16, 16)]
        lv = v - c0
        ok = (lv >= 0) & (lv < N_HALF)
        idx_v[row, pl.ds(t * 16, 16)] = jnp.where(ok, lv, dummy)


def _gather_rows(table, idx3):
    @functools.partial(
        pl.kernel,
        out_type=jax.ShapeDtypeStruct((S_PAD, D), jnp.float32),
        mesh=_sc_mesh(),
        scratch_types=[
            pltpu.VMEM((1, CH), jnp.int32),
            pltpu.VMEM((CH, D), jnp.float32),
            pltpu.SemaphoreType.DMA,
        ],
    )
    def gk(table_hbm, idx_hbm, out_hbm, idx_v, rows_v, sem):
        wid = lax.axis_index("s") * NC + lax.axis_index("c")
        pltpu.sync_copy(idx_hbm.at[wid], idx_v)
        pltpu.async_copy(table_hbm.at[idx_v.at[0]], rows_v, sem).wait()
        pltpu.sync_copy(rows_v, out_hbm.at[pl.ds(wid * CH, CH)])

    return gk(table, idx3)


def _scatter_block(u, msg, dst3, nsrc3):
    out_type = [jax.ShapeDtypeStruct((N_PAD, D), jnp.float32)]
    scratch = [
        pltpu.VMEM_SHARED((ACC_ROWS, D), jnp.float32),
        pltpu.VMEM((2, CH), jnp.int32),
        pltpu.VMEM((2, CH, D), jnp.float32),
        pltpu.SemaphoreType.DMA,
    ]
    with_gather = nsrc3 is not None
    if with_gather:
        out_type += [
            jax.ShapeDtypeStruct((S_PAD, D), jnp.float32),
            jax.ShapeDtypeStruct((S_PAD, D), jnp.float32),
        ]
        scratch += [
            pltpu.VMEM((2, CH), jnp.int32),
            pltpu.VMEM((2 * CH, D), jnp.float32),
            pltpu.VMEM((8, D), jnp.float32),
            pltpu.SemaphoreType.DMA,
        ]

    @functools.partial(
        pl.kernel, out_type=out_type, mesh=_sc_mesh(), scratch_types=scratch
    )
    def sk(*refs):
        if with_gather:
            (u_hbm, msg_hbm, dst_hbm, nsrc_hbm, p_hbm, g0_hbm, g1_hbm,
             acc_sh, idx_v, msg_v, sem, nidx_v, grows_v, zbuf, gsem) = refs
        else:
            (u_hbm, msg_hbm, dst_hbm, p_hbm,
             acc_sh, idx_v, msg_v, sem) = refs
        cid = lax.axis_index("c")
        sid = lax.axis_index("s")
        c0 = cid * N_HALF
        r0 = sid * TILE_ROWS

        pltpu.sync_copy(
            u_hbm.at[pl.ds(c0 + r0, TILE_ROWS)], acc_sh.at[pl.ds(r0, TILE_ROWS)]
        )
        pltpu.sync_copy(dst_hbm.at[sid], idx_v)
        for j in range(2):
            _clamp_to_half(idx_v, j, c0, DUMMY_SCAT)
            pltpu.sync_copy(
                msg_hbm.at[pl.ds(sid * 2 * CH + j * CH, CH)], msg_v.at[j]
            )
        if with_gather:
            @pl.when(sid == 0)
            def _zero_miss_rows():
                for r in range(8):
                    for t in range(D // 16):
                        zbuf[r, pl.ds(t * 16, 16)] = jnp.zeros((16,), jnp.float32)
                pltpu.sync_copy(zbuf, acc_sh.at[pl.ds(DUMMY_GATH, 8)])

        plsc.subcore_barrier()

        for j in range(2):
            pltpu.sync_copy(msg_v.at[j], acc_sh.at[idx_v.at[j]], add=True)

        plsc.subcore_barrier()

        wb = pltpu.async_copy(
            acc_sh.at[pl.ds(r0, TILE_ROWS)],
            p_hbm.at[pl.ds(c0 + r0, TILE_ROWS)],
            sem,
        )
        if with_gather:
            pltpu.sync_copy(nsrc_hbm.at[sid], nidx_v)
            for j in range(2):
                _clamp_to_half(nidx_v, j, c0, DUMMY_GATH)
                pltpu.async_copy(
                    acc_sh.at[nidx_v.at[j]],
                    grows_v.at[pl.ds(j * CH, CH)],
                    gsem,
                ).wait()

            @pl.when(cid == 0)
            def _out0():
                pltpu.sync_copy(grows_v, g0_hbm.at[pl.ds(sid * 2 * CH, 2 * CH)])

            @pl.when(cid == 1)
            def _out1():
                pltpu.sync_copy(grows_v, g1_hbm.at[pl.ds(sid * 2 * CH, 2 * CH)])

        wb.wait()

    return sk(u, msg, dst3, nsrc3) if with_gather else sk(u, msg, dst3)


M_PLANES = [2, 4, 6, 0]


def _mm_block(g_parts, subea_t, p, wmsg, wedge, wself, b2, use_relu):
    n_rows = p.shape[0]
    two_g = len(g_parts) == 2

    def body(*refs):
        if two_g:
            (g0_ref, g1_ref, se_ref, p_ref, wm_ref, we_ref, ws_ref, b_ref,
             msg_out, u_out) = refs
            gg = g0_ref[...] + g1_ref[...]
        else:
            (g_ref, se_ref, p_ref, wm_ref, we_ref, ws_ref, b_ref,
             msg_out, u_out) = refs
            gg = g_ref[...]
        pp = p_ref[...]
        if use_relu:
            gg = jnp.maximum(gg, 0.0)
            pp = jnp.maximum(pp, 0.0)
        e_k = lax.dot_general(
            se_ref[...], we_ref[...], (((0,), (0,)), ((), ())),
            preferred_element_type=jnp.float32,
        )
        msg_out[...] = (
            jnp.dot(gg, wm_ref[...], preferred_element_type=jnp.float32)
            + e_k
        )
        u_out[pl.ds(0, n_rows), :] = (
            jnp.dot(pp, ws_ref[...], preferred_element_type=jnp.float32)
            + b_ref[...]
        )
        if n_rows < N_PAD:
            u_out[pl.ds(n_rows, N_PAD - n_rows), :] = jnp.zeros(
                (N_PAD - n_rows, D), jnp.float32
            )

    return pl.pallas_call(
        body,
        out_shape=[
            jax.ShapeDtypeStruct((S_PAD, D), jnp.float32),
            jax.ShapeDtypeStruct((N_PAD, D), jnp.float32),
        ],
    )(*g_parts, subea_t, p, wmsg, wedge, wself, b2)


def _relu_kernel(p):
    def body(p_ref, o_ref):
        o_ref[...] = jnp.maximum(p_ref[pl.ds(0, N_NODES), :], 0.0)

    return pl.pallas_call(
        body, out_shape=jax.ShapeDtypeStruct((N_NODES, D), jnp.float32)
    )(p)


def kernel(x, ei, ea, batch, y, W_msg, W_edge, W_self, b):
    base = np.arange(0, N_NODES, NB)
    k2 = np.stack(
        [(2 * ((base + k) % N_NODES)) % N_EDGES for k in range(1, NB + 1)]
    ).astype(np.int32)
    for ki, m in enumerate(M_PLANES):
        if not np.array_equal(np.sort(k2[ki]), np.arange(S) * 8 + m):
            raise AssertionError("static edge-subset pattern mismatch")

    cols = lax.slice(ei, (0, 0), (2, 8 * S)).reshape(2, S, 8)
    srcs, dsts = [], []
    pad_i = jnp.zeros((S_PAD - S,), jnp.int32)
    pad_d = jnp.full((S_PAD - S,), DUMMY_DST, jnp.int32)
    for m in M_PLANES:
        srcs.append(jnp.concatenate([cols[0, :, m], pad_i]))
        dsts.append(jnp.concatenate([cols[1, :, m], pad_d]))
    src = jnp.stack(srcs)
    dst = jnp.stack(dsts)

    src_w = src.reshape(NB, NW, 1, CH)
    src_t = src.reshape(NB, NS, 2, CH)
    dst3 = dst.reshape(NB, NS, 2, CH)
    b2 = b.reshape(1, D)

    ea_t = ea.T
    subea_t = [
        lax.slice(ea_t, (0, m), (D_EDGE, 8 * S_PAD), (1, 8))
        for m in M_PLANES
    ]

    p_cur = x
    g_parts = (_gather_rows(x, src_w[0]),)
    for k in range(NB):
        msg, u = _mm_block(g_parts, subea_t[k], p_cur, W_msg, W_edge, W_self,
                           b2, use_relu=(k > 0))
        nsrc = src_t[k + 1] if k + 1 < NB else None
        res = _scatter_block(u, msg, dst3[k], nsrc)
        if k + 1 < NB:
            p_cur, g0, g1 = res
            g_parts = (g0, g1)
        else:
            (p_cur,) = res

    return _relu_kernel(p_cur)
